# baseline (speedup 1.0000x reference)
import functools

import jax
import jax.numpy as jnp
from jax import lax
from jax.experimental import pallas as pl
from jax.experimental.pallas import tpu as pltpu
from jax.experimental.pallas import tpu_sc as plsc

_IN = 4096
_OUT = 4096
_K = 1228
_BO = 512

_SC_R = 512
_S = _OUT - _SC_R
_NW = 32
_R_PER = _SC_R // _NW
_G = 8
_NGRP = _R_PER // _G


def _select_body(x_ref, xm_ref):
    xv = x_ref[...]
    s = jnp.abs(xv)
    bits = jax.lax.bitcast_convert_type(s, jnp.int32)
    j16 = jax.lax.broadcasted_iota(jnp.int32, (16, 1), 0)

    t = jnp.int32(0)
    for p in range(8):
        shift = 28 - 4 * p
        cand = t + (j16 << shift)
        cnts = jnp.sum((bits >= cand).astype(jnp.int32), axis=1, keepdims=True)
        ok = (cnts >= _K) & (cand >= 0)
        d = jnp.sum(ok.astype(jnp.int32)) - 1
        t = t + (d << shift)

    gt = bits > t
    eq = bits == t
    r = _K - jnp.sum(gt.astype(jnp.int32))
    iota = jax.lax.broadcasted_iota(jnp.int32, (1, _IN), 1)
    eq_i = eq.astype(jnp.int32)

    pfx = jnp.int32(0)
    for p in range(3):
        shift = 8 - 4 * p
        cand = pfx + (j16 << shift)
        f = jnp.sum(jnp.where(iota < cand, eq_i, 0), axis=1, keepdims=True)
        d = jnp.maximum(jnp.sum((f < r).astype(jnp.int32)) - 1, 0)
        pfx = pfx + (d << shift)
    istar = jnp.where(r > 0, pfx + 1, 0)

    mask = gt | (eq & (iota < istar))
    xm_ref[...] = jnp.where(mask, xv, 0.0)


def _mm_body(xm_ref, w_ref, b_ref, o_ref):
    acc = jax.lax.dot_general(
        xm_ref[...], w_ref[...], (((1,), (1,)), ((), ())),
        preferred_element_type=jnp.float32,
    )
    o_ref[...] = acc + b_ref[...]


def _lane_sum(v):
    lane = lax.iota(jnp.int32, 16)
    for sh in (8, 4, 2, 1):
        perm = jnp.bitwise_xor(lane, sh)
        v = v + v.at[perm].get(mode="promise_in_bounds", unique_indices=True)
    return v


_sc_mesh = plsc.VectorSubcoreMesh(core_axis_name="c", subcore_axis_name="s")


@functools.partial(
    pl.kernel,
    mesh=_sc_mesh,
    out_type=jax.ShapeDtypeStruct((1, _SC_R), jnp.float32),
    scratch_types=[
        pltpu.VMEM((1, _IN), jnp.float32),
        pltpu.VMEM((_G, _IN), jnp.float32),
        pltpu.VMEM((_G, _IN), jnp.float32),
        pltpu.VMEM((_R_PER,), jnp.float32),
        pltpu.VMEM((_R_PER,), jnp.float32),
        pltpu.SemaphoreType.DMA,
        pltpu.SemaphoreType.DMA,
    ],
)
def _sc_matvec(xm_hbm, w_hbm, b_hbm, out_hbm,
               xm_v, bufa, bufb, acc_v, bias_v, sema, semb):
    cid = lax.axis_index("c")
    sid = lax.axis_index("s")
    wid = sid * 2 + cid
    base = _S + wid * _R_PER
    obase = wid * _R_PER

    pltpu.sync_copy(xm_hbm, xm_v)
    pltpu.sync_copy(b_hbm.at[0, pl.ds(base, _R_PER)], bias_v)

    lane = lax.iota(jnp.int32, 16)
    bufs = (bufa, bufb)
    sems = (sema, semb)

    cp = pltpu.async_copy(w_hbm.at[pl.ds(base, _G), :], bufa, sema)
    for gi in range(_NGRP):
        buf = bufs[gi % 2]
        if gi + 1 < _NGRP:
            cp_next = pltpu.async_copy(
                w_hbm.at[pl.ds(base + (gi + 1) * _G, _G), :],
                bufs[(gi + 1) % 2], sems[(gi + 1) % 2])
        cp.wait()

        zero = jnp.zeros((16,), jnp.float32)

        def chunk(ci, accs):
            off = ci * 16
            xc = xm_v[0, pl.ds(off, 16)]
            return tuple(
                accs[j] + buf[j, pl.ds(off, 16)] * xc for j in range(_G)
            )

        accs = lax.fori_loop(0, _IN // 16, chunk, (zero,) * _G)

        out_chunk = jnp.zeros((16,), jnp.float32)
        for j in range(_G):
            red = _lane_sum(accs[j])
            lane_j = (gi * _G + j) % 16
            out_chunk = out_chunk + jnp.where(
                lane == lane_j, red, jnp.float32(0.0))
        t = (gi * _G) // 16
        half = (gi * _G) % 16
        if half == 0:
            pending = out_chunk
        else:
            full = pending + out_chunk
            acc_v[pl.ds(t * 16, 16)] = full + bias_v[pl.ds(t * 16, 16)]
        if gi + 1 < _NGRP:
            cp = cp_next

    pltpu.sync_copy(acc_v, out_hbm.at[0, pl.ds(obase, _R_PER)])


@jax.jit
def _run(x2, w, b2):
    xm = pl.pallas_call(
        _select_body,
        out_shape=jax.ShapeDtypeStruct((1, _IN), jnp.float32),
    )(x2)

    tc_out = pl.pallas_call(
        _mm_body,
        grid=(_S // _BO,),
        in_specs=[
            pl.BlockSpec((1, _IN), lambda g: (0, 0)),
            pl.BlockSpec((_BO, _IN), lambda g: (g, 0)),
            pl.BlockSpec((1, _BO), lambda g: (0, g)),
        ],
        out_specs=pl.BlockSpec((1, _BO), lambda g: (0, g)),
        out_shape=jax.ShapeDtypeStruct((1, _S), jnp.float32),
    )(xm, w, b2)

    sc_out = _sc_matvec(xm, w, b2)
    return jnp.concatenate([tc_out, sc_out], axis=1)


def kernel(x, weight, bias):
    bsz, seq, _ = x.shape
    out = _run(x.reshape(1, _IN), weight, bias.reshape(1, _OUT))
    return out.reshape(bsz, seq, _OUT)


# speedup vs baseline: 1.7710x; 1.7710x over previous
import jax
import jax.numpy as jnp
from jax.experimental import pallas as pl
from jax.experimental.pallas import tpu as pltpu

_IN = 4096
_OUT = 4096
_K = 1228
_BO = 512
_NB = _OUT // _BO


def _do_select(x_ref, xm_ref):
    xv = x_ref[...]
    s = jnp.abs(xv)
    bits = jax.lax.bitcast_convert_type(s, jnp.int32)
    j16 = jax.lax.broadcasted_iota(jnp.int32, (16, 1), 0)

    t = jnp.int32(0)
    for p in range(8):
        shift = 28 - 4 * p
        cand = t + (j16 << shift)
        cnts = jnp.sum((bits >= cand).astype(jnp.int32), axis=1, keepdims=True)
        ok = (cnts >= _K) & (cand >= 0)
        d = jnp.sum(ok.astype(jnp.int32)) - 1
        t = t + (d << shift)

    gt = bits > t
    eq = bits == t
    r = _K - jnp.sum(gt.astype(jnp.int32))
    iota = jax.lax.broadcasted_iota(jnp.int32, (1, _IN), 1)
    eq_i = eq.astype(jnp.int32)

    pfx = jnp.int32(0)
    for p in range(3):
        shift = 8 - 4 * p
        cand = pfx + (j16 << shift)
        f = jnp.sum(jnp.where(iota < cand, eq_i, 0), axis=1, keepdims=True)
        d = jnp.maximum(jnp.sum((f < r).astype(jnp.int32)) - 1, 0)
        pfx = pfx + (d << shift)
    istar = jnp.where(r > 0, pfx + 1, 0)

    mask = gt | (eq & (iota < istar))
    xm_ref[...] = jnp.where(mask, xv, 0.0)


def _body(x_ref, w_ref, b_ref, o_ref, xm_ref):
    g = pl.program_id(0)

    @pl.when(g == 0)
    def _select():
        _do_select(x_ref, xm_ref)

    @pl.when(g > 0)
    def _mm():
        acc = jax.lax.dot_general(
            xm_ref[...], w_ref[...], (((1,), (1,)), ((), ())),
            preferred_element_type=jnp.float32,
        )
        o_ref[...] = acc + b_ref[...]


@jax.jit
def _run(x2, w, b2):
    def _wmap(g):
        return (jnp.maximum(g - 1, 0), 0)

    def _omap(g):
        return (0, jnp.maximum(g - 1, 0))

    return pl.pallas_call(
        _body,
        grid=(_NB + 1,),
        in_specs=[
            pl.BlockSpec((1, _IN), lambda g: (0, 0)),
            pl.BlockSpec((_BO, _IN), _wmap),
            pl.BlockSpec((1, _BO), _omap),
        ],
        out_specs=pl.BlockSpec((1, _BO), _omap),
        out_shape=jax.ShapeDtypeStruct((1, _OUT), jnp.float32),
        scratch_shapes=[pltpu.VMEM((1, _IN), jnp.float32)],
    )(x2, w, b2)


def kernel(x, weight, bias):
    bsz, seq, _ = x.shape
    out = _run(x.reshape(1, _IN), weight, bias.reshape(1, _OUT))
    return out.reshape(bsz, seq, _OUT)
